# Initial kernel scaffold; baseline (speedup 1.0000x reference)
import jax
import jax.numpy as jnp
from jax.experimental import pallas as pl


def kernel(idx, targets, embedding_table):
    raise NotImplementedError("write your pallas kernel here")


# speedup vs baseline: 1.0235x; 1.0235x over previous
import functools

import jax
import jax.numpy as jnp
from jax import lax
from jax.experimental import pallas as pl
from jax.experimental.pallas import tpu as pltpu
from jax.experimental.pallas import tpu_sc as plsc

NUM_CORES = 2
NUM_SUBCORES = 16
NW = NUM_CORES * NUM_SUBCORES

EMB_DIM = 32
TOTAL_ROWS = 16384 * 50
B_PER_W = TOTAL_ROWS // NW
CHUNK = 128
NSTEPS = B_PER_W // CHUNK


def _gather_body(idx_hbm, table_hbm, out_hbm, idx_v, rows_v, gsem):
    wid = lax.axis_index("s") * NUM_CORES + lax.axis_index("c")
    base = wid * B_PER_W
    pltpu.sync_copy(idx_hbm.at[wid], idx_v)

    @pl.loop(0, NSTEPS)
    def _step(j):
        pltpu.async_copy(table_hbm.at[idx_v.at[j]], rows_v, gsem).wait()
        pltpu.sync_copy(rows_v, out_hbm.at[pl.ds(base + j * CHUNK, CHUNK)])


@functools.partial(jax.jit, static_argnames=())
def _lookup(idx_flat, table):
    mesh = plsc.VectorSubcoreMesh(core_axis_name="c", subcore_axis_name="s")
    return pl.kernel(
        _gather_body,
        out_type=jax.ShapeDtypeStruct((TOTAL_ROWS, EMB_DIM), jnp.float32),
        mesh=mesh,
        compiler_params=pltpu.CompilerParams(use_tc_tiling_on_sc=False),
        scratch_types=[
            pltpu.VMEM((NSTEPS, CHUNK), jnp.int32),
            pltpu.VMEM((CHUNK, EMB_DIM), jnp.float32),
            pltpu.SemaphoreType.DMA,
        ],
    )(idx_flat, table)


def kernel(idx, targets, embedding_table):
    del targets
    idx_flat = idx.astype(jnp.int32).reshape(NW, NSTEPS, CHUNK)
    out = _lookup(idx_flat, embedding_table)
    return out.reshape(idx.shape[0], idx.shape[1], EMB_DIM)


# speedup vs baseline: 1.1126x; 1.0871x over previous
import functools

import jax
import jax.numpy as jnp
from jax import lax
from jax.experimental import pallas as pl
from jax.experimental.pallas import tpu as pltpu
from jax.experimental.pallas import tpu_sc as plsc

NUM_CORES = 2
NUM_SUBCORES = 16
NW = NUM_CORES * NUM_SUBCORES

EMB_DIM = 32
TOTAL_ROWS = 16384 * 50
B_PER_W = TOTAL_ROWS // NW
CHUNK = 128
NSTEPS = B_PER_W // CHUNK


NBUF = 8


def _gather_body(idx_hbm, table_hbm, out_hbm, idx_v, *scratch):
    rows = scratch[:NBUF]
    gsem = scratch[NBUF:2 * NBUF]
    ssem = scratch[2 * NBUF:3 * NBUF]
    wid = lax.axis_index("s") * NUM_CORES + lax.axis_index("c")
    base = wid * B_PER_W
    pltpu.sync_copy(idx_hbm.at[wid], idx_v)

    def start_gather(j, b):
        pltpu.async_copy(table_hbm.at[idx_v.at[j]], rows[b], gsem[b])

    def wait_gather(j, b):
        pltpu.make_async_copy(table_hbm.at[idx_v.at[j]], rows[b], gsem[b]).wait()

    def out_slice(j):
        return out_hbm.at[pl.ds(base + j * CHUNK, CHUNK)]

    def start_store(j, b):
        pltpu.async_copy(rows[b], out_slice(j), ssem[b])

    def wait_store(j, b):
        pltpu.make_async_copy(rows[b], out_slice(j), ssem[b]).wait()

    for b in range(NBUF):
        start_gather(b, b)

    @pl.loop(0, NSTEPS - NBUF, step=NBUF)
    def _round(g):
        for b in range(NBUF):
            j = g + b
            wait_gather(j, b)
            start_store(j, b)
        for b in range(NBUF):
            wait_store(g + b, b)
            start_gather(g + NBUF + b, b)

    for b in range(NBUF):
        j = NSTEPS - NBUF + b
        wait_gather(j, b)
        start_store(j, b)
    for b in range(NBUF):
        wait_store(NSTEPS - NBUF + b, b)


@functools.partial(jax.jit, static_argnames=())
def _lookup(idx_flat, table):
    mesh = plsc.VectorSubcoreMesh(core_axis_name="c", subcore_axis_name="s")
    return pl.kernel(
        _gather_body,
        out_type=jax.ShapeDtypeStruct((TOTAL_ROWS, EMB_DIM), jnp.float32),
        mesh=mesh,
        compiler_params=pltpu.CompilerParams(use_tc_tiling_on_sc=False),
        scratch_types=(
            [pltpu.VMEM((NSTEPS, CHUNK), jnp.int32)]
            + [pltpu.VMEM((CHUNK, EMB_DIM), jnp.float32) for _ in range(NBUF)]
            + [pltpu.SemaphoreType.DMA for _ in range(2 * NBUF)]
        ),
    )(idx_flat, table)


def kernel(idx, targets, embedding_table):
    del targets
    idx_flat = idx.astype(jnp.int32).reshape(NW, NSTEPS, CHUNK)
    out = _lookup(idx_flat, embedding_table)
    return out.reshape(idx.shape[0], idx.shape[1], EMB_DIM)


# speedup vs baseline: 1.5487x; 1.3919x over previous
import functools

import jax
import jax.numpy as jnp
from jax import lax
from jax.experimental import pallas as pl
from jax.experimental.pallas import tpu as pltpu
from jax.experimental.pallas import tpu_sc as plsc

NUM_CORES = 2
NUM_SUBCORES = 16
NW = NUM_CORES * NUM_SUBCORES

VOCAB = 1_000_000
EMB = 32
SEQ = 50
BATCH = 16384
B_PER_W = BATCH // NW

SROWS = VOCAB // 4

CHUNK = 128
N_CH2 = SEQ * (B_PER_W // CHUNK)
GBUF = 4


def _gather_body(idx_hbm, staged, out_hbm, idx_v, qv_list, gb_list,
                 tt_a, tt_b, sem_q, gsem_list, osem_a, osem_b):
    cid = lax.axis_index("c")
    sid = lax.axis_index("s")
    wid = sid * NUM_CORES + cid
    lanes = lax.iota(jnp.int32, 16)

    pltpu.async_copy(
        idx_hbm.at[:, pl.ds(wid * B_PER_W, B_PER_W)], idx_v, sem_q).wait()

    def build_q(m, slot):
        s = lax.shift_right_logical(m, 2)
        cc = lax.bitwise_and(m, 3)
        for k0 in range(0, CHUNK, 16):
            r = idx_v[s, pl.ds(cc * CHUNK + k0, 16)]
            qv_list[slot][pl.ds(k0, 16)] = lax.shift_right_logical(r, 2)

    def start_gather(slot):
        pltpu.async_copy(staged.at[qv_list[slot]], gb_list[slot],
                         gsem_list[slot])

    def wait_gather(slot):
        pltpu.make_async_copy(staged.at[qv_list[slot]], gb_list[slot],
                              gsem_list[slot]).wait()

    def extract(m, slot, tt_v):
        s = lax.shift_right_logical(m, 2)
        cc = lax.bitwise_and(m, 3)
        gb = gb_list[slot]
        for k0 in range(0, CHUNK, 16):
            r = idx_v[s, pl.ds(cc * CHUNK + k0, 16)]
            col0 = lax.shift_left(lax.bitwise_and(r, 3), 5)
            kv = k0 + lanes
            for d in range(EMB):
                tt_v[d, pl.ds(k0, 16)] = plsc.load_gather(gb, [kv, col0 + d])

    def out_ref(m):
        s = lax.shift_right_logical(m, 2)
        cc = lax.bitwise_and(m, 3)
        return out_hbm.at[s, :, pl.ds(wid * B_PER_W + cc * CHUNK, CHUNK)]

    def wait_store(tt_v, osem):
        pltpu.make_async_copy(
            tt_v, out_hbm.at[0, :, pl.ds(0, CHUNK)], osem).wait()

    for b in range(GBUF):
        build_q(jnp.int32(b), b)
        start_gather(b)

    tts = (tt_a, tt_b)
    osems = (osem_a, osem_b)

    @pl.loop(0, N_CH2, step=GBUF)
    def _p2(m):
        for j in range(GBUF):
            mj = m + j
            tb = j % 2

            if j >= 2:
                wait_store(tts[tb], osems[tb])
            else:
                @pl.when(m > 0)
                def _():
                    wait_store(tts[tb], osems[tb])

            wait_gather(j)
            extract(mj, j, tts[tb])
            nxt = mj + GBUF

            @pl.when(nxt < N_CH2)
            def _():
                build_q(nxt, j)
                start_gather(j)

            pltpu.async_copy(tts[tb], out_ref(mj), osems[tb])

    wait_store(tt_a, osem_a)
    wait_store(tt_b, osem_b)


@jax.jit
def _lookup(idx_t, staged):
    mesh = plsc.VectorSubcoreMesh(core_axis_name="c", subcore_axis_name="s")
    return pl.kernel(
        _gather_body,
        out_type=jax.ShapeDtypeStruct((SEQ, EMB, BATCH), jnp.float32),
        mesh=mesh,
        compiler_params=pltpu.CompilerParams(
            use_tc_tiling_on_sc=True, needs_layout_passes=False),
        scratch_types=(
            [pltpu.VMEM((SEQ, B_PER_W), jnp.int32)]
            + [[pltpu.VMEM((CHUNK,), jnp.int32) for _ in range(GBUF)]]
            + [[pltpu.VMEM((CHUNK, 128), jnp.float32) for _ in range(GBUF)]]
            + [pltpu.VMEM((EMB, CHUNK), jnp.float32) for _ in range(2)]
            + [pltpu.SemaphoreType.DMA]
            + [[pltpu.SemaphoreType.DMA for _ in range(GBUF)]]
            + [pltpu.SemaphoreType.DMA for _ in range(2)]
        ),
    )(idx_t, staged)


def kernel(idx, targets, embedding_table):
    del targets
    idx_t = idx.astype(jnp.int32).T
    staged = embedding_table.reshape(SROWS, 128)
    out_t = _lookup(idx_t, staged)
    return jnp.transpose(out_t, (2, 0, 1))


# speedup vs baseline: 2.4018x; 1.5508x over previous
import functools

import jax
import jax.numpy as jnp
from jax import lax
from jax.experimental import pallas as pl
from jax.experimental.pallas import tpu as pltpu
from jax.experimental.pallas import tpu_sc as plsc

NUM_CORES = 2
NUM_SUBCORES = 16
NW = NUM_CORES * NUM_SUBCORES

VOCAB = 1_000_000
EMB = 32
SEQ = 50
BATCH = 16384
B_PER_W = BATCH // NW

SROWS = VOCAB // 4

CHUNK = 128
N_CH2 = SEQ * (B_PER_W // CHUNK)
GBUF = 2


def _gather_body(idx_hbm, staged, out_hbm, idx_v, qv_list, gb_list,
                 tt_a, tt_b, sem_q, gsem_list, osem_a, osem_b):
    cid = lax.axis_index("c")
    sid = lax.axis_index("s")
    wid = sid * NUM_CORES + cid
    lanes = lax.iota(jnp.int32, 16)

    pltpu.async_copy(
        idx_hbm.at[:, pl.ds(wid * B_PER_W, B_PER_W)], idx_v, sem_q).wait()

    def build_q(m, slot):
        s = lax.shift_right_logical(m, 2)
        cc = lax.bitwise_and(m, 3)

        @pl.loop(0, CHUNK // 16)
        def _(kk):
            k0 = kk * 16
            r = idx_v[s, pl.ds(cc * CHUNK + k0, 16)]
            qv_list[slot][pl.ds(k0, 16)] = lax.shift_right_logical(r, 2)

    def start_gather(slot):
        pltpu.async_copy(staged.at[qv_list[slot]], gb_list[slot],
                         gsem_list[slot])

    def wait_gather(slot):
        pltpu.make_async_copy(staged.at[qv_list[slot]], gb_list[slot],
                              gsem_list[slot]).wait()

    diags = [lax.bitwise_and(lanes + t, 15) for t in range(16)]

    def extract(m, slot, tt_v):
        s = lax.shift_right_logical(m, 2)
        cc = lax.bitwise_and(m, 3)
        gb = gb_list[slot]

        @pl.loop(0, CHUNK // 16)
        def _(kk):
            k0 = kk * 16
            r = idx_v[s, pl.ds(cc * CHUNK + k0, 16)]
            col0 = lax.shift_left(lax.bitwise_and(r, 3), 5)
            kv = k0 + lanes
            for d0 in (0, 16):
                for t in range(16):
                    dvec = d0 + diags[t]
                    vec = plsc.load_gather(gb, [kv, col0 + dvec])
                    plsc.store_scatter(tt_v, [dvec, kv], vec)

    def out_ref(m):
        s = lax.shift_right_logical(m, 2)
        cc = lax.bitwise_and(m, 3)
        return out_hbm.at[s, :, pl.ds(wid * B_PER_W + cc * CHUNK, CHUNK)]

    def wait_store(tt_v, osem):
        pltpu.make_async_copy(
            tt_v, out_hbm.at[0, :, pl.ds(0, CHUNK)], osem).wait()

    for b in range(GBUF):
        build_q(jnp.int32(b), b)
        start_gather(b)

    tts = (tt_a, tt_b)
    osems = (osem_a, osem_b)

    @pl.loop(0, N_CH2, step=GBUF)
    def _p2(m):
        for j in range(GBUF):
            mj = m + j
            tb = j % 2

            if j >= 2:
                wait_store(tts[tb], osems[tb])
            else:
                @pl.when(m > 0)
                def _():
                    wait_store(tts[tb], osems[tb])

            wait_gather(j)
            extract(mj, j, tts[tb])
            nxt = mj + GBUF

            @pl.when(nxt < N_CH2)
            def _():
                build_q(nxt, j)
                start_gather(j)

            pltpu.async_copy(tts[tb], out_ref(mj), osems[tb])

    wait_store(tt_a, osem_a)
    wait_store(tt_b, osem_b)


@jax.jit
def _lookup(idx_t, staged):
    mesh = plsc.VectorSubcoreMesh(core_axis_name="c", subcore_axis_name="s")
    return pl.kernel(
        _gather_body,
        out_type=jax.ShapeDtypeStruct((SEQ, EMB, BATCH), jnp.float32),
        mesh=mesh,
        compiler_params=pltpu.CompilerParams(
            use_tc_tiling_on_sc=True, needs_layout_passes=False),
        scratch_types=(
            [pltpu.VMEM((SEQ, B_PER_W), jnp.int32)]
            + [[pltpu.VMEM((CHUNK,), jnp.int32) for _ in range(GBUF)]]
            + [[pltpu.VMEM((CHUNK, 128), jnp.float32) for _ in range(GBUF)]]
            + [pltpu.VMEM((EMB, CHUNK), jnp.float32) for _ in range(2)]
            + [pltpu.SemaphoreType.DMA]
            + [[pltpu.SemaphoreType.DMA for _ in range(GBUF)]]
            + [pltpu.SemaphoreType.DMA for _ in range(2)]
        ),
    )(idx_t, staged)


def kernel(idx, targets, embedding_table):
    del targets
    idx_t = idx.astype(jnp.int32).T
    staged = embedding_table.reshape(SROWS, 128)
    out_t = _lookup(idx_t, staged)
    return jnp.transpose(out_t, (2, 0, 1))


# speedup vs baseline: 3.4468x; 1.4351x over previous
import functools

import jax
import jax.numpy as jnp
from jax import lax
from jax.experimental import pallas as pl
from jax.experimental.pallas import tpu as pltpu
from jax.experimental.pallas import tpu_sc as plsc

NUM_CORES = 2
NUM_SUBCORES = 16
NW = NUM_CORES * NUM_SUBCORES

VOCAB = 1_000_000
EMB = 32
SEQ = 50
BATCH = 16384
B_PER_W = BATCH // NW

SROWS = VOCAB // 4

STCH = 128
N_STCH = VOCAB // STCH
STAIL = VOCAB - N_STCH * STCH

CHUNK = 128
N_CH2 = SEQ * (B_PER_W // CHUNK)
GBUF = 2


def _stage_body(table_hbm, tail_hbm, staged_hbm, tc_a, tc_b, st_a, st_b, tl_v,
                lsem_a, lsem_b, ssem_a, ssem_b):
    cid = lax.axis_index("c")
    sid = lax.axis_index("s")
    wid = sid * NUM_CORES + cid
    lanes = lax.iota(jnp.int32, 16)
    diags = [lax.bitwise_and(lanes + t, 15) for t in range(16)]

    tcs, sts = (tc_a, tc_b), (st_a, st_b)
    lsems, ssems = (lsem_a, lsem_b), (ssem_a, ssem_b)

    n_mine = (N_STCH - wid + NW - 1) // NW

    def start_load(c, j):
        pltpu.async_copy(table_hbm.at[:, pl.ds(c * STCH, STCH)], tcs[j],
                         lsems[j])

    def wait_load(j):
        pltpu.make_async_copy(table_hbm.at[:, pl.ds(0, STCH)], tcs[j],
                              lsems[j]).wait()

    def start_store(c, j):
        pltpu.async_copy(sts[j], staged_hbm.at[pl.ds(c * (STCH // 4),
                                                     STCH // 4)], ssems[j])

    def wait_store(j):
        pltpu.make_async_copy(sts[j], staged_hbm.at[pl.ds(0, STCH // 4)],
                              ssems[j]).wait()

    def transpose(src, dst, width):
        @pl.loop(0, width // 16)
        def _(ii):
            iv = ii * 16 + lanes
            qloc = lax.shift_right_logical(iv, 2)
            colbase = lax.shift_left(lax.bitwise_and(iv, 3), 5)
            for d0 in (0, 16):
                for t in range(16):
                    dvec = d0 + diags[t]
                    vec = plsc.load_gather(src, [dvec, iv])
                    plsc.store_scatter(dst, [qloc, colbase + dvec], vec)

    start_load(wid, 0)

    @pl.loop(0, n_mine, step=2)
    def _run(t):
        for j in range(2):
            tj = t + j

            @pl.when(tj < n_mine)
            def _():
                wait_load(j)

                @pl.when(tj + 1 < n_mine)
                def _():
                    start_load(wid + NW * (tj + 1), 1 - j)

                @pl.when(tj >= 2)
                def _():
                    wait_store(j)

                transpose(tcs[j], sts[j], STCH)
                start_store(wid + NW * tj, j)

    for j in range(2):
        @pl.when(n_mine > j)
        def _():
            wait_store(j)

    @pl.when(wid == NW - 1)
    def _tail():
        pltpu.async_copy(tail_hbm, tl_v, lsem_a).wait()
        transpose(tl_v, st_a, STAIL)
        pltpu.async_copy(
            st_a.at[pl.ds(0, STAIL // 4)],
            staged_hbm.at[pl.ds(N_STCH * (STCH // 4), STAIL // 4)],
            ssem_a).wait()


@jax.jit
def _stage(table_t, tail_t):
    mesh = plsc.VectorSubcoreMesh(core_axis_name="c", subcore_axis_name="s")
    return pl.kernel(
        _stage_body,
        out_type=jax.ShapeDtypeStruct((SROWS, 128), jnp.float32),
        mesh=mesh,
        compiler_params=pltpu.CompilerParams(
            use_tc_tiling_on_sc=True, needs_layout_passes=False),
        scratch_types=(
            [pltpu.VMEM((EMB, STCH), jnp.float32) for _ in range(2)]
            + [pltpu.VMEM((STCH // 4, 128), jnp.float32) for _ in range(2)]
            + [pltpu.VMEM((EMB, STAIL), jnp.float32)]
            + [pltpu.SemaphoreType.DMA for _ in range(4)]
        ),
    )(table_t, tail_t)


def _gather_body(idx_hbm, staged, out_hbm, idx_v, qv_list, gb_list,
                 tt_a, tt_b, sem_q, gsem_list, osem_a, osem_b):
    cid = lax.axis_index("c")
    sid = lax.axis_index("s")
    wid = sid * NUM_CORES + cid
    lanes = lax.iota(jnp.int32, 16)

    pltpu.async_copy(
        idx_hbm.at[:, pl.ds(wid * B_PER_W, B_PER_W)], idx_v, sem_q).wait()

    def build_q(m, slot):
        s = lax.shift_right_logical(m, 2)
        cc = lax.bitwise_and(m, 3)

        @pl.loop(0, CHUNK // 16)
        def _(kk):
            k0 = kk * 16
            r = idx_v[s, pl.ds(cc * CHUNK + k0, 16)]
            qv_list[slot][pl.ds(k0, 16)] = lax.shift_right_logical(r, 2)

    def start_gather(slot):
        pltpu.async_copy(staged.at[qv_list[slot]], gb_list[slot],
                         gsem_list[slot])

    def wait_gather(slot):
        pltpu.make_async_copy(staged.at[qv_list[slot]], gb_list[slot],
                              gsem_list[slot]).wait()

    diags = [lax.bitwise_and(lanes + t, 15) for t in range(16)]

    def extract(m, slot, tt_v):
        s = lax.shift_right_logical(m, 2)
        cc = lax.bitwise_and(m, 3)
        gb = gb_list[slot]

        @pl.loop(0, CHUNK // 16)
        def _(kk):
            k0 = kk * 16
            r = idx_v[s, pl.ds(cc * CHUNK + k0, 16)]
            col0 = lax.shift_left(lax.bitwise_and(r, 3), 5)
            kv = k0 + lanes
            for d0 in (0, 16):
                for t in range(16):
                    dvec = d0 + diags[t]
                    vec = plsc.load_gather(gb, [kv, col0 + dvec])
                    plsc.store_scatter(tt_v, [dvec, kv], vec)

    def out_ref(m):
        s = lax.shift_right_logical(m, 2)
        cc = lax.bitwise_and(m, 3)
        return out_hbm.at[s, :, pl.ds(wid * B_PER_W + cc * CHUNK, CHUNK)]

    def wait_store(tt_v, osem):
        pltpu.make_async_copy(
            tt_v, out_hbm.at[0, :, pl.ds(0, CHUNK)], osem).wait()

    for b in range(GBUF):
        build_q(jnp.int32(b), b)
        start_gather(b)

    tts = (tt_a, tt_b)
    osems = (osem_a, osem_b)

    @pl.loop(0, N_CH2, step=GBUF)
    def _p2(m):
        for j in range(GBUF):
            mj = m + j
            tb = j % 2

            if j >= 2:
                wait_store(tts[tb], osems[tb])
            else:
                @pl.when(m > 0)
                def _():
                    wait_store(tts[tb], osems[tb])

            wait_gather(j)
            extract(mj, j, tts[tb])
            nxt = mj + GBUF

            @pl.when(nxt < N_CH2)
            def _():
                build_q(nxt, j)
                start_gather(j)

            pltpu.async_copy(tts[tb], out_ref(mj), osems[tb])

    wait_store(tt_a, osem_a)
    wait_store(tt_b, osem_b)


@jax.jit
def _lookup(idx_t, staged):
    mesh = plsc.VectorSubcoreMesh(core_axis_name="c", subcore_axis_name="s")
    return pl.kernel(
        _gather_body,
        out_type=jax.ShapeDtypeStruct((SEQ, EMB, BATCH), jnp.float32),
        mesh=mesh,
        compiler_params=pltpu.CompilerParams(
            use_tc_tiling_on_sc=True, needs_layout_passes=False),
        scratch_types=(
            [pltpu.VMEM((SEQ, B_PER_W), jnp.int32)]
            + [[pltpu.VMEM((CHUNK,), jnp.int32) for _ in range(GBUF)]]
            + [[pltpu.VMEM((CHUNK, 128), jnp.float32) for _ in range(GBUF)]]
            + [pltpu.VMEM((EMB, CHUNK), jnp.float32) for _ in range(2)]
            + [pltpu.SemaphoreType.DMA]
            + [[pltpu.SemaphoreType.DMA for _ in range(GBUF)]]
            + [pltpu.SemaphoreType.DMA for _ in range(2)]
        ),
    )(idx_t, staged)


def kernel(idx, targets, embedding_table):
    del targets
    idx_t = idx.astype(jnp.int32).T
    table_t = embedding_table.T
    tail_t = embedding_table[N_STCH * STCH:].T
    staged = _stage(table_t, tail_t)
    out_t = _lookup(idx_t, staged)
    return jnp.transpose(out_t, (2, 0, 1))
